# Initial kernel scaffold; baseline (speedup 1.0000x reference)
import jax
import jax.numpy as jnp
from jax.experimental import pallas as pl


def kernel(x, weight, row_mask_u):
    raise NotImplementedError("write your pallas kernel here")


# speedup vs baseline: 1.1184x; 1.1184x over previous
import functools
import jax
import jax.numpy as jnp
from jax import lax
from jax.experimental import pallas as pl
from jax.experimental.pallas import tpu as pltpu
from jax.experimental.pallas import tpu_sc as plsc

DROP_P = 0.1
KEEP = 1.0 - DROP_P
SCALE = 1.0 / KEEP

NC = 2
NS = 16
NW = NC * NS
L = 16

B = 4096 * 50
D = 64
CH = 128
BPW = B // NW
NCHUNK = BPW // CH


def _sc_body(w_hbm, u_hbm, x_hbm, out_hbm, idx_v, u_v, rows_v, sem_u, sem_r):
    cid = lax.axis_index("c")
    sid = lax.axis_index("s")
    wid = sid * NC + cid
    pltpu.sync_copy(x_hbm.at[pl.ds(wid * BPW, BPW)], idx_v)
    out_base = wid * BPW

    def chunk(j, carry):
        idxs = idx_v.at[pl.ds(j * CH, CH)]
        cp_u = pltpu.async_copy(u_hbm.at[idxs], u_v, sem_u)
        cp_r = pltpu.async_copy(w_hbm.at[idxs], rows_v, sem_r)
        cp_u.wait()
        cp_r.wait()
        for g in range(CH // L):
            u16 = u_v[pl.ds(g * L, L)]
            s16 = jnp.where(u16 < KEEP, jnp.float32(SCALE), jnp.float32(0.0))
            for r in range(L):
                row = g * L + r
                sv = jnp.full((L,), s16[r], jnp.float32)
                for cg in range(D // L):
                    sl = pl.ds(cg * L, L)
                    rows_v[row, sl] = rows_v[row, sl] * sv
        pltpu.sync_copy(rows_v, out_hbm.at[pl.ds(out_base + j * CH, CH)])
        return carry

    lax.fori_loop(0, NCHUNK, chunk, 0)


@jax.jit
def _embedding_dropout(x_flat, weight, u_flat):
    mesh = plsc.VectorSubcoreMesh(
        core_axis_name="c", subcore_axis_name="s", num_cores=NC, num_subcores=NS
    )
    fn = pl.kernel(
        _sc_body,
        out_type=jax.ShapeDtypeStruct((B, D), jnp.float32),
        mesh=mesh,
        scratch_types=[
            pltpu.VMEM((BPW,), jnp.int32),
            pltpu.VMEM((CH,), jnp.float32),
            pltpu.VMEM((CH, D), jnp.float32),
            pltpu.SemaphoreType.DMA,
            pltpu.SemaphoreType.DMA,
        ],
        compiler_params=pltpu.CompilerParams(use_tc_tiling_on_sc=False),
    )
    return fn(weight, u_flat, x_flat)


def kernel(x, weight, row_mask_u):
    x_flat = x.reshape(-1).astype(jnp.int32)
    u_flat = row_mask_u.reshape(-1)
    out = _embedding_dropout(x_flat, weight, u_flat)
    return out.reshape(x.shape[0], x.shape[1], D)
